# baseline (speedup 1.0000x reference)
import jax
import jax.numpy as jnp
from jax import lax
from jax.experimental import pallas as pl
from jax.experimental.pallas import tpu as pltpu
from jax.experimental.pallas import tpu_sc as plsc

_NC = 2
_NS = 16
_NW = _NC * _NS
_L = 16
_B = 16384
_D = 32
_BPW = _B // _NW
_CH = 64


def _tec_body(ent_hbm, rel_hbm, ph_hbm, pr_hbm, pt_hbm, nh_hbm, nr_hbm, nt_hbm,
              out_hbm,
              iv0, iv1, iv2, iv3, iv4, iv5,
              is0, is1, is2, is3, is4, is5,
              b0, b1, b2, b3, b4, b5,
              s3, s4, s5, acc_v, sems):
    sid = lax.axis_index("s")
    wid = sid * _NC + lax.axis_index("c")
    base = wid * _BPW

    srcs = (ph_hbm, pr_hbm, pt_hbm, nh_hbm, nr_hbm, nt_hbm)
    tabs = (ent_hbm, rel_hbm, ent_hbm, ent_hbm, rel_hbm, ent_hbm)
    ivs = (iv0, iv1, iv2, iv3, iv4, iv5)
    iss = (is0, is1, is2, is3, is4, is5)
    bufs = (b0, b1, b2, b3, b4, b5)
    spms = (s3, s4, s5)

    for t in range(6):
        pltpu.sync_copy(srcs[t].at[pl.ds(base, _BPW)],
                        ivs[t].at[pl.ds(sid * _BPW, _BPW)])

    acc = jnp.zeros((_L,), jnp.float32)
    for ci in range(_BPW // _CH):
        for t in range(6):
            pltpu.sync_copy(ivs[t].at[pl.ds(sid * _BPW + ci * _CH, _CH)],
                            iss[t])

        def row(i, c):
            for t in range(3):
                pltpu.async_copy(
                    tabs[t].at[pl.ds(iss[t][i], 1), :],
                    bufs[t].at[pl.ds(i, 1), :],
                    sems.at[t])
            for t in range(3, 6):
                pltpu.async_copy(
                    tabs[t].at[pl.ds(iss[t][i], 1), :],
                    spms[t - 3].at[pl.ds(sid * _CH + i, 1), :],
                    sems.at[t])
            return c

        lax.fori_loop(0, _CH, row, 0)

        for t in range(3):
            pltpu.make_async_copy(tabs[t], bufs[t], sems.at[t]).wait()
        for t in range(3, 6):
            pltpu.make_async_copy(
                tabs[t], spms[t - 3].at[pl.ds(sid * _CH, _CH), :],
                sems.at[t]).wait()

        for t in range(3, 6):
            pltpu.sync_copy(spms[t - 3].at[pl.ds(sid * _CH, _CH), :],
                            bufs[t])

        def step(r, a):
            for c in (0, _L):
                p = jnp.abs(b0[r, pl.ds(c, _L)] + b1[r, pl.ds(c, _L)]
                            - b2[r, pl.ds(c, _L)])
                n = jnp.abs(b3[r, pl.ds(c, _L)] + b4[r, pl.ds(c, _L)]
                            - b5[r, pl.ds(c, _L)])
                a = a + (n - p)
            return a

        acc = lax.fori_loop(0, _CH, step, acc)

    acc_v[...] = acc
    pltpu.sync_copy(acc_v, out_hbm.at[wid])


@jax.jit
def kernel(pos_exmpl, neg_exmpl, entity_emb, relation_emb):
    mesh = plsc.VectorSubcoreMesh(core_axis_name="c", subcore_axis_name="s")
    partials = pl.kernel(
        _tec_body,
        out_type=jax.ShapeDtypeStruct((_NW, _L), jnp.float32),
        mesh=mesh,
        scratch_types=(
            [pltpu.VMEM_SHARED((_NS * _BPW,), jnp.int32)] * 6
            + [pltpu.SMEM((_CH,), jnp.int32)] * 6
            + [pltpu.VMEM((_CH, _D), jnp.float32)] * 6
            + [pltpu.VMEM_SHARED((_NS * _CH, _D), jnp.float32)] * 3
            + [pltpu.VMEM((_L,), jnp.float32), pltpu.SemaphoreType.DMA((6,))]
        ),
        compiler_params=pltpu.CompilerParams(use_tc_tiling_on_sc=True),
    )(entity_emb, relation_emb,
      pos_exmpl[0].astype(jnp.int32), pos_exmpl[1].astype(jnp.int32),
      pos_exmpl[2].astype(jnp.int32), neg_exmpl[0].astype(jnp.int32),
      neg_exmpl[1].astype(jnp.int32), neg_exmpl[2].astype(jnp.int32))
    return jnp.maximum(0.0, jnp.sum(partials) + 1.0)


# speedup vs baseline: 1.1251x; 1.1251x over previous
import jax
import jax.numpy as jnp
from jax import lax
from jax.experimental import pallas as pl
from jax.experimental.pallas import tpu as pltpu
from jax.experimental.pallas import tpu_sc as plsc

_NC = 2
_NS = 16
_NW = _NC * _NS
_L = 16
_B = 16384
_D = 32
_BPW = _B // _NW
_CH = 128


def _tec_body(ent_hbm, rel_hbm, ph_hbm, pr_hbm, pt_hbm, nh_hbm, nr_hbm, nt_hbm,
              out_hbm,
              iv0, iv1, iv2, iv3, iv4, iv5,
              is0, is1, is2, is3, is4, is5,
              b0, b1, b2, b3, b4, b5, acc_v, sem):
    sid = lax.axis_index("s")
    wid = sid * _NC + lax.axis_index("c")
    base = wid * _BPW

    srcs = (ph_hbm, pr_hbm, pt_hbm, nh_hbm, nr_hbm, nt_hbm)
    tabs = (ent_hbm, rel_hbm, ent_hbm, ent_hbm, rel_hbm, ent_hbm)
    ivs = (iv0, iv1, iv2, iv3, iv4, iv5)
    iss = (is0, is1, is2, is3, is4, is5)
    bufs = (b0, b1, b2, b3, b4, b5)

    for t in range(6):
        pltpu.sync_copy(srcs[t].at[pl.ds(base, _BPW)],
                        ivs[t].at[pl.ds(sid * _BPW, _BPW)])

    acc = jnp.zeros((_L,), jnp.float32)
    for ci in range(_BPW // _CH):
        for t in range(6):
            pltpu.sync_copy(ivs[t].at[pl.ds(sid * _BPW + ci * _CH, _CH)],
                            iss[t])

        def row(i, c):
            for t in range(6):
                pltpu.async_copy(
                    tabs[t].at[pl.ds(iss[t][i], 1), :],
                    bufs[t].at[pl.ds(i, 1), :],
                    sem)
            return c

        lax.fori_loop(0, _CH, row, 0)

        for t in range(6):
            pltpu.make_async_copy(tabs[t], bufs[t], sem).wait()

        def step(r, a):
            for c in (0, _L):
                p = jnp.abs(b0[r, pl.ds(c, _L)] + b1[r, pl.ds(c, _L)]
                            - b2[r, pl.ds(c, _L)])
                n = jnp.abs(b3[r, pl.ds(c, _L)] + b4[r, pl.ds(c, _L)]
                            - b5[r, pl.ds(c, _L)])
                a = a + (n - p)
            return a

        acc = lax.fori_loop(0, _CH, step, acc)

    acc_v[...] = acc
    pltpu.sync_copy(acc_v, out_hbm.at[wid])


@jax.jit
def kernel(pos_exmpl, neg_exmpl, entity_emb, relation_emb):
    mesh = plsc.VectorSubcoreMesh(core_axis_name="c", subcore_axis_name="s")
    partials = pl.kernel(
        _tec_body,
        out_type=jax.ShapeDtypeStruct((_NW, _L), jnp.float32),
        mesh=mesh,
        scratch_types=(
            [pltpu.VMEM_SHARED((_NS * _BPW,), jnp.int32)] * 6
            + [pltpu.SMEM((_CH,), jnp.int32)] * 6
            + [pltpu.VMEM((_CH, _D), jnp.float32)] * 6
            + [pltpu.VMEM((_L,), jnp.float32), pltpu.SemaphoreType.DMA]
        ),
        compiler_params=pltpu.CompilerParams(use_tc_tiling_on_sc=True),
    )(entity_emb, relation_emb,
      pos_exmpl[0].astype(jnp.int32), pos_exmpl[1].astype(jnp.int32),
      pos_exmpl[2].astype(jnp.int32), neg_exmpl[0].astype(jnp.int32),
      neg_exmpl[1].astype(jnp.int32), neg_exmpl[2].astype(jnp.int32))
    return jnp.maximum(0.0, jnp.sum(partials) + 1.0)
